# baseline (speedup 1.0000x reference)
import functools

import jax
import jax.numpy as jnp
from jax import lax
from jax.experimental import pallas as pl
from jax.experimental.pallas import tpu as pltpu
from jax.experimental.pallas import tpu_sc as plsc

_D = 64
_L = 16
_NC, _NS = 2, 16
_NW = _NC * _NS
_BT = 128
_SB = 2
_NSLOT = 3

_SEQ = 200
_BATCH = 4096
_NBT = _BATCH // _BT
_NBLK = _SEQ // _SB


def _embed_lookup(x2, table):
    mesh = plsc.VectorSubcoreMesh(core_axis_name="c", subcore_axis_name="s")

    @functools.partial(
        pl.kernel,
        out_type=jax.ShapeDtypeStruct((_SEQ, _D // 8, _NBT, 8 * _BT), jnp.float32),
        mesh=mesh,
        scratch_types=[
            pltpu.VMEM((_SEQ, _BT), jnp.int32),
            pltpu.VMEM((_NSLOT, _SB, _BT, _D), jnp.float32),
            pltpu.VMEM((_NSLOT, _SB, _D // 8, 8 * _BT), jnp.float32),
            pltpu.SemaphoreType.DMA,
            pltpu.SemaphoreType.DMA,
            pltpu.SemaphoreType.DMA,
            pltpu.SemaphoreType.DMA,
            pltpu.SemaphoreType.DMA,
            pltpu.SemaphoreType.DMA,
            pltpu.SemaphoreType.DMA,
        ],
        compiler_params=pltpu.CompilerParams(use_tc_tiling_on_sc=False,
                                             needs_layout_passes=False),
    )
    def k(x_hbm, table_hbm, out_hbm, idx_all, rows_v, tbuf, isem,
          g0, g1, g2, o0, o1, o2):
        gsem = (g0, g1, g2)
        osem = (o0, o1, o2)
        w = lax.axis_index("s") * _NC + lax.axis_index("c")
        ii = lax.iota(jnp.int32, _L)

        pltpu.async_copy(x_hbm.at[:, pl.ds(w * _BT, _BT)], idx_all, isem).wait()

        def fire_gather(blk, slot):
            for j in range(_SB):
                pltpu.async_copy(table_hbm.at[idx_all.at[blk * _SB + j]],
                                 rows_v.at[slot, j], gsem[slot])

        def wait_gather(blk, slot):
            for j in range(_SB):
                pltpu.make_async_copy(table_hbm.at[idx_all.at[blk * _SB + j]],
                                      rows_v.at[slot, j], gsem[slot]).wait()

        def transpose_block(slot):
            @plsc.parallel_loop(0, _SB * _D, 1, unroll=2)
            def per_e(je):
                j = je // _D
                e = je % _D
                evec = jnp.full((_L,), 0, jnp.int32) + e
                col = (e % 8) * _BT
                vs = [plsc.load_gather(rows_v.at[slot, j], [ii + g * _L, evec])
                      for g in range(_BT // _L)]
                for g in range(_BT // _L):
                    tbuf[slot, j, e // 8, pl.ds(col + g * _L, _L)] = vs[g] * 8.0

        def fire_out(blk, slot):
            for j in range(_SB):
                pltpu.async_copy(tbuf.at[slot, j],
                                 out_hbm.at[blk * _SB + j, :, w], osem[slot])

        def wait_out(blk, slot):
            for j in range(_SB):
                pltpu.make_async_copy(tbuf.at[slot, j],
                                      out_hbm.at[blk * _SB + j, :, w],
                                      osem[slot]).wait()

        fire_gather(0, 0)
        fire_gather(1, 1)

        def step(blk, slot, pslot):
            @pl.when(blk + 2 < _NBLK)
            def _():
                @pl.when(blk + 2 >= _NSLOT)
                def _():
                    wait_out(blk + 2 - _NSLOT, pslot)

                fire_gather(blk + 2, pslot)

            wait_gather(blk, slot)
            transpose_block(slot)
            fire_out(blk, slot)

        def outer(o, carry):
            for k_ in range(_NSLOT):
                blk = o * _NSLOT + k_
                step(blk, k_, (k_ + 2) % _NSLOT)
            return carry

        lax.fori_loop(0, _NBLK // _NSLOT, outer, 0)
        step(_NBLK - 1, (_NBLK - 1) % _NSLOT, (_NBLK + 1) % _NSLOT)

        for k_ in range(_NSLOT):
            wait_out(_NBLK - _NSLOT + k_, (_NBLK - _NSLOT + k_) % _NSLOT)

    return k(x2, table)


def kernel(x, table):
    out4 = _embed_lookup(x.T, table)
    out = (out4.reshape(_SEQ, _D // 8, _NBT, 8, _BT)
           .transpose(2, 4, 0, 1, 3)
           .reshape(_BATCH, _SEQ, _D))
    return out


# speedup vs baseline: 1.5994x; 1.5994x over previous
import functools

import jax
import jax.numpy as jnp
from jax import lax
from jax.experimental import pallas as pl
from jax.experimental.pallas import tpu as pltpu
from jax.experimental.pallas import tpu_sc as plsc

_D = 64
_L = 16
_NC, _NS = 2, 16
_NW = _NC * _NS
_BT = 128
_SB = 2
_NSLOT = 3
_PITCH = 65

_SEQ = 200
_BATCH = 4096
_NBT = _BATCH // _BT
_NBLK = _SEQ // _SB
_NCHUNK = 8


def _embed_lookup(x2, table):
    mesh = plsc.VectorSubcoreMesh(core_axis_name="c", subcore_axis_name="s")
    crows = _SEQ // _NCHUNK

    @functools.partial(
        pl.kernel,
        out_type=jax.ShapeDtypeStruct((_SEQ, _D // 8, _NBT, 8 * _BT), jnp.float32),
        mesh=mesh,
        scratch_types=[
            pltpu.VMEM((crows, _BT), jnp.int32),
            pltpu.VMEM((_SEQ, _BT), jnp.int32),
            pltpu.VMEM((_NSLOT, _SB, _BT, _D), jnp.float32),
            pltpu.VMEM((_NSLOT, _SB, _D // 8, 8 * _BT), jnp.float32),
            pltpu.SemaphoreType.DMA,
            pltpu.SemaphoreType.DMA,
            pltpu.SemaphoreType.DMA,
            pltpu.SemaphoreType.DMA,
            pltpu.SemaphoreType.DMA,
            pltpu.SemaphoreType.DMA,
            pltpu.SemaphoreType.DMA,
        ],
        compiler_params=pltpu.CompilerParams(use_tc_tiling_on_sc=False,
                                             needs_layout_passes=False),
    )
    def k(x_hbm, table_hbm, out_hbm, idx_raw, idx_all, rows_v, tbuf, isem,
          g0, g1, g2, o0, o1, o2):
        gsem = (g0, g1, g2)
        osem = (o0, o1, o2)
        w = lax.axis_index("s") * _NC + lax.axis_index("c")
        ii = lax.iota(jnp.int32, _L)
        ii200 = ii * 200

        def stage_chunk(c, carry):
            pltpu.async_copy(x_hbm.at[pl.ds(w * _SEQ + c * crows, crows)],
                             idx_raw, isem).wait()

            @plsc.parallel_loop(0, _SEQ, 1, unroll=4)
            def per_s(s):
                fl = ii200 + s
                v = plsc.load_gather(idx_raw, [fl >> 7, fl & 127])
                idx_all[s, pl.ds(c * _L, _L)] = v

            return carry

        lax.fori_loop(0, _NCHUNK, stage_chunk, 0)

        def fire_gather(blk, slot):
            for j in range(_SB):
                pltpu.async_copy(table_hbm.at[idx_all.at[blk * _SB + j]],
                                 rows_v.at[slot, j], gsem[slot])

        def wait_gather(blk, slot):
            for j in range(_SB):
                pltpu.make_async_copy(table_hbm.at[idx_all.at[blk * _SB + j]],
                                      rows_v.at[slot, j], gsem[slot]).wait()

        rot = [(ii + r) & 15 for r in range(_L)]

        def transpose_block(slot):
            @plsc.parallel_loop(0, _SB * 32, 1, unroll=1)
            def per_tile(t):
                j = t // 32
                bg = (t % 32) // 4
                eg = t % 4
                bvec = ii + bg * _L
                for r in range(_L):
                    evec = rot[r] + eg * _L
                    v = plsc.load_gather(rows_v.at[slot, j], [bvec, evec])
                    rowv = evec >> 3
                    colv = ((evec & 7) << 7) + bvec
                    plsc.store_scatter(tbuf.at[slot, j], [rowv, colv], v * 8.0)

        def fire_out(blk, slot):
            for j in range(_SB):
                pltpu.async_copy(tbuf.at[slot, j],
                                 out_hbm.at[blk * _SB + j, :, w], osem[slot])

        def wait_out(blk, slot):
            for j in range(_SB):
                pltpu.make_async_copy(tbuf.at[slot, j],
                                      out_hbm.at[blk * _SB + j, :, w],
                                      osem[slot]).wait()

        fire_gather(0, 0)
        fire_gather(1, 1)

        def step(blk, slot, pslot):
            @pl.when(blk + 2 < _NBLK)
            def _():
                @pl.when(blk + 2 >= _NSLOT)
                def _():
                    wait_out(blk + 2 - _NSLOT, pslot)

                fire_gather(blk + 2, pslot)

            wait_gather(blk, slot)
            transpose_block(slot)
            fire_out(blk, slot)

        def outer(o, carry):
            for k_ in range(_NSLOT):
                blk = o * _NSLOT + k_
                step(blk, k_, (k_ + 2) % _NSLOT)
            return carry

        lax.fori_loop(0, _NBLK // _NSLOT, outer, 0)
        step(_NBLK - 1, (_NBLK - 1) % _NSLOT, (_NBLK + 1) % _NSLOT)

        for k_ in range(_NSLOT):
            wait_out(_NBLK - _NSLOT + k_, (_NBLK - _NSLOT + k_) % _NSLOT)

    return k(x2, table)


def kernel(x, table):
    out4 = _embed_lookup(x.reshape(_SEQ * _NBT, _BT), table)
    out = (out4.reshape(_SEQ, _D // 8, _NBT, 8, _BT)
           .transpose(2, 4, 0, 1, 3)
           .reshape(_BATCH, _SEQ, _D))
    return out


# speedup vs baseline: 1.6183x; 1.0118x over previous
import functools

import jax
import jax.numpy as jnp
from jax import lax
from jax.experimental import pallas as pl
from jax.experimental.pallas import tpu as pltpu
from jax.experimental.pallas import tpu_sc as plsc

_D = 64
_L = 16
_NC, _NS = 2, 16
_NW = _NC * _NS
_BT = 128
_NSLOT = 3

_SEQ = 200
_BATCH = 4096
_NBT = _BATCH // _BT
_NCHUNK = 8


def _embed_lookup(x2, t2):
    mesh = plsc.VectorSubcoreMesh(core_axis_name="c", subcore_axis_name="s")
    crows = _SEQ // _NCHUNK

    @functools.partial(
        pl.kernel,
        out_type=jax.ShapeDtypeStruct((_SEQ, _D // 8, _NBT, 8 * _BT), jnp.float32),
        mesh=mesh,
        scratch_types=[
            pltpu.VMEM((crows, _BT), jnp.int32),
            pltpu.VMEM((_SEQ, _BT), jnp.int32),
            pltpu.VMEM((_NSLOT, _BT), jnp.int32),
            pltpu.VMEM((_NSLOT, _BT, 2 * _D), jnp.float32),
            pltpu.VMEM((_NSLOT, _D // 8, 8 * _BT), jnp.float32),
            pltpu.SemaphoreType.DMA,
            pltpu.SemaphoreType.DMA,
            pltpu.SemaphoreType.DMA,
            pltpu.SemaphoreType.DMA,
            pltpu.SemaphoreType.DMA,
            pltpu.SemaphoreType.DMA,
            pltpu.SemaphoreType.DMA,
        ],
        compiler_params=pltpu.CompilerParams(use_tc_tiling_on_sc=False,
                                             needs_layout_passes=False),
    )
    def k(x_hbm, t2_hbm, out_hbm, idx_raw, idx_all, vp_slot, rows_v, tbuf,
          isem, g0, g1, g2, o0, o1, o2):
        gsem = (g0, g1, g2)
        osem = (o0, o1, o2)
        w = lax.axis_index("s") * _NC + lax.axis_index("c")
        ii = lax.iota(jnp.int32, _L)
        ii200 = ii * 200

        def stage_chunk(c, carry):
            pltpu.async_copy(x_hbm.at[pl.ds(w * _SEQ + c * crows, crows)],
                             idx_raw, isem).wait()

            @plsc.parallel_loop(0, _SEQ, 1, unroll=4)
            def per_s(s):
                fl = ii200 + s
                v = plsc.load_gather(idx_raw, [fl >> 7, fl & 127])
                idx_all[s, pl.ds(c * _L, _L)] = v

            return carry

        lax.fori_loop(0, _NCHUNK, stage_chunk, 0)

        def fire_gather(blk, slot):
            for g in range(_BT // _L):
                iv = idx_all[blk, pl.ds(g * _L, _L)]
                vp_slot[slot, pl.ds(g * _L, _L)] = iv >> 1
            pltpu.async_copy(t2_hbm.at[vp_slot.at[slot]], rows_v.at[slot],
                             gsem[slot])

        def wait_gather(slot):
            pltpu.make_async_copy(t2_hbm.at[vp_slot.at[slot]],
                                  rows_v.at[slot], gsem[slot]).wait()

        rot = [(ii + r) & 15 for r in range(_L)]
        rowadd = [r >> 3 for r in rot]
        coladd = [(r & 7) << 7 for r in rot]

        def transpose_block(blk, slot):
            @plsc.parallel_loop(0, 32, 1, unroll=1)
            def per_tile(t):
                bg = t // 4
                eg = t % 4
                bvec = ii + bg * _L
                iv = idx_all[blk, pl.ds(bg * _L, _L)]
                ebase = ((iv & 1) << 6) + eg * _L
                eg2 = eg * 2
                for r in range(_L):
                    v = plsc.load_gather(rows_v.at[slot], [bvec, ebase + rot[r]])
                    plsc.store_scatter(tbuf.at[slot],
                                       [rowadd[r] + eg2, coladd[r] + bvec],
                                       v * 8.0)

        def fire_out(blk, slot):
            pltpu.async_copy(tbuf.at[slot], out_hbm.at[blk, :, w], osem[slot])

        def wait_out(blk, slot):
            pltpu.make_async_copy(tbuf.at[slot], out_hbm.at[blk, :, w],
                                  osem[slot]).wait()

        fire_gather(0, 0)
        fire_gather(1, 1)

        def step(blk, slot, pslot):
            @pl.when(blk + 2 < _SEQ)
            def _():
                @pl.when(blk + 2 >= _NSLOT)
                def _():
                    wait_out(blk + 2 - _NSLOT, pslot)

                fire_gather(blk + 2, pslot)

            wait_gather(slot)
            transpose_block(blk, slot)
            fire_out(blk, slot)

        def outer(o, carry):
            for k_ in range(_NSLOT):
                blk = o * _NSLOT + k_
                step(blk, k_, (k_ + 2) % _NSLOT)
            return carry

        lax.fori_loop(0, _SEQ // _NSLOT, outer, 0)
        step(_SEQ - 2, (_SEQ - 2) % _NSLOT, _SEQ % _NSLOT)
        step(_SEQ - 1, (_SEQ - 1) % _NSLOT, (_SEQ + 1) % _NSLOT)

        for k_ in range(_NSLOT):
            wait_out(_SEQ - _NSLOT + k_, (_SEQ - _NSLOT + k_) % _NSLOT)

    return k(x2, t2)


def kernel(x, table):
    out4 = _embed_lookup(x.reshape(_SEQ * _NBT, _BT),
                         table.reshape(table.shape[0] // 2, 2 * _D))
    out = (out4.reshape(_SEQ, _D // 8, _NBT, 8, _BT)
           .transpose(2, 4, 0, 1, 3)
           .reshape(_BATCH, _SEQ, _D))
    return out


# speedup vs baseline: 1.6572x; 1.0240x over previous
import functools

import jax
import jax.numpy as jnp
from jax import lax
from jax.experimental import pallas as pl
from jax.experimental.pallas import tpu as pltpu
from jax.experimental.pallas import tpu_sc as plsc

_D = 64
_L = 16
_NC, _NS = 2, 16
_NW = _NC * _NS
_BT = 128
_NSLOT = 3

_SEQ = 200
_BATCH = 4096
_NBT = _BATCH // _BT
_NCHUNK = 8


def _embed_lookup(x2, t2):
    mesh = plsc.VectorSubcoreMesh(core_axis_name="c", subcore_axis_name="s")
    crows = _SEQ // _NCHUNK

    @functools.partial(
        pl.kernel,
        out_type=jax.ShapeDtypeStruct((_SEQ, _D // 8, _NBT, 8 * _BT), jnp.float32),
        mesh=mesh,
        scratch_types=[
            pltpu.VMEM((crows, _BT), jnp.int32),
            pltpu.VMEM((_SEQ, _BT), jnp.int32),
            pltpu.VMEM((_NSLOT, _BT, 2 * _D), jnp.float32),
            pltpu.VMEM((_NSLOT, _D // 8, 8 * _BT), jnp.float32),
            pltpu.SemaphoreType.DMA,
            pltpu.SemaphoreType.DMA,
            pltpu.SemaphoreType.DMA,
            pltpu.SemaphoreType.DMA,
            pltpu.SemaphoreType.DMA,
            pltpu.SemaphoreType.DMA,
            pltpu.SemaphoreType.DMA,
        ],
        compiler_params=pltpu.CompilerParams(use_tc_tiling_on_sc=False,
                                             needs_layout_passes=False),
    )
    def k(x_hbm, t2_hbm, out_hbm, idx_raw, idx_all, rows_v, tbuf,
          isem, g0, g1, g2, o0, o1, o2):
        gsem = (g0, g1, g2)
        osem = (o0, o1, o2)
        w = lax.axis_index("s") * _NC + lax.axis_index("c")
        ii = lax.iota(jnp.int32, _L)
        ii200 = ii * 200

        def stage_chunk(c, carry):
            pltpu.async_copy(x_hbm.at[pl.ds(w * _SEQ + c * crows, crows)],
                             idx_raw, isem).wait()

            @plsc.parallel_loop(0, _SEQ, 1, unroll=4)
            def per_s(s):
                fl = ii200 + s
                v = plsc.load_gather(idx_raw, [fl >> 7, fl & 127])
                idx_all[s, pl.ds(c * _L, _L)] = v

            return carry

        lax.fori_loop(0, _NCHUNK, stage_chunk, 0)

        def fire_gather(blk, slot):
            pltpu.async_copy(t2_hbm.at[idx_all.at[blk]], rows_v.at[slot],
                             gsem[slot])

        def wait_gather(blk, slot):
            pltpu.make_async_copy(t2_hbm.at[idx_all.at[blk]],
                                  rows_v.at[slot], gsem[slot]).wait()

        rot = [(ii + r) & 15 for r in range(_L)]
        rowadd = [r >> 3 for r in rot]
        coladd = [(r & 7) << 7 for r in rot]

        def transpose_block(blk, slot):
            @plsc.parallel_loop(0, 32, 1, unroll=1)
            def per_tile(t):
                bg = t // 4
                eg = t % 4
                bvec = ii + bg * _L
                ebase = jnp.full((_L,), 0, jnp.int32) + eg * _L
                eg2 = eg * 2
                for r in range(_L):
                    v = plsc.load_gather(rows_v.at[slot], [bvec, ebase + rot[r]])
                    plsc.store_scatter(tbuf.at[slot],
                                       [rowadd[r] + eg2, coladd[r] + bvec],
                                       v * 8.0)

        def fire_out(blk, slot):
            pltpu.async_copy(tbuf.at[slot], out_hbm.at[blk, :, w], osem[slot])

        def wait_out(blk, slot):
            pltpu.make_async_copy(tbuf.at[slot], out_hbm.at[blk, :, w],
                                  osem[slot]).wait()

        fire_gather(0, 0)
        fire_gather(1, 1)

        def step(blk, slot, pslot):
            @pl.when(blk + 2 < _SEQ)
            def _():
                @pl.when(blk + 2 >= _NSLOT)
                def _():
                    wait_out(blk + 2 - _NSLOT, pslot)

                fire_gather(blk + 2, pslot)

            wait_gather(blk, slot)
            transpose_block(blk, slot)
            fire_out(blk, slot)

        def outer(o, carry):
            for k_ in range(_NSLOT):
                blk = o * _NSLOT + k_
                step(blk, k_, (k_ + 2) % _NSLOT)
            return carry

        lax.fori_loop(0, _SEQ // _NSLOT, outer, 0)
        step(_SEQ - 2, (_SEQ - 2) % _NSLOT, _SEQ % _NSLOT)
        step(_SEQ - 1, (_SEQ - 1) % _NSLOT, (_SEQ + 1) % _NSLOT)

        for k_ in range(_NSLOT):
            wait_out(_SEQ - _NSLOT + k_, (_SEQ - _NSLOT + k_) % _NSLOT)

    return k(x2, t2)


def kernel(x, table):
    tpad = jnp.pad(table, ((0, 0), (0, 2 * _D - table.shape[1])))
    out4 = _embed_lookup(x.reshape(_SEQ * _NBT, _BT), tpad)
    out = (out4.reshape(_SEQ, _D // 8, _NBT, 8, _BT)
           .transpose(2, 4, 0, 1, 3)
           .reshape(_BATCH, _SEQ, _D))
    return out


# speedup vs baseline: 1.6607x; 1.0021x over previous
import functools

import jax
import jax.numpy as jnp
from jax import lax
from jax.experimental import pallas as pl
from jax.experimental.pallas import tpu as pltpu
from jax.experimental.pallas import tpu_sc as plsc

_D = 64
_L = 16
_NC, _NS = 2, 16
_NW = _NC * _NS
_BT = 128
_NSLOT = 3

_SEQ = 200
_BATCH = 4096
_NBT = _BATCH // _BT
_NCHUNK = 8


def _embed_lookup(x2, t2):
    mesh = plsc.VectorSubcoreMesh(core_axis_name="c", subcore_axis_name="s")
    crows = _SEQ // _NCHUNK

    @functools.partial(
        pl.kernel,
        out_type=jax.ShapeDtypeStruct((_SEQ, _D // 8, _NBT, 8 * _BT), jnp.float32),
        mesh=mesh,
        scratch_types=[
            pltpu.VMEM((crows, _BT), jnp.int32),
            pltpu.VMEM((_SEQ, _BT), jnp.int32),
            pltpu.VMEM((_NSLOT, _BT, 2 * _D), jnp.float32),
            pltpu.VMEM((_NSLOT, _D // 8, 8 * _BT), jnp.float32),
            pltpu.SemaphoreType.DMA,
            pltpu.SemaphoreType.DMA,
            pltpu.SemaphoreType.DMA,
            pltpu.SemaphoreType.DMA,
            pltpu.SemaphoreType.DMA,
            pltpu.SemaphoreType.DMA,
            pltpu.SemaphoreType.DMA,
        ],
        compiler_params=pltpu.CompilerParams(use_tc_tiling_on_sc=False,
                                             needs_layout_passes=False),
    )
    def k(x_hbm, t2_hbm, out_hbm, idx_raw, idx_all, rows_v, tbuf,
          isem, g0, g1, g2, o0, o1, o2):
        gsem = (g0, g1, g2)
        osem = (o0, o1, o2)
        w = lax.axis_index("s") * _NC + lax.axis_index("c")
        ii = lax.iota(jnp.int32, _L)
        ii200 = ii * 200

        def stage_chunk(c, carry):
            pltpu.async_copy(x_hbm.at[pl.ds(w * _SEQ + c * crows, crows)],
                             idx_raw, isem).wait()

            @plsc.parallel_loop(0, _SEQ, 1, unroll=4)
            def per_s(s):
                fl = ii200 + s
                v = plsc.load_gather(idx_raw, [fl >> 7, fl & 127])
                idx_all[s, pl.ds(c * _L, _L)] = v

            return carry

        lax.fori_loop(0, _NCHUNK, stage_chunk, 0)

        def fire_gather(blk, slot):
            pltpu.async_copy(t2_hbm.at[idx_all.at[blk]], rows_v.at[slot],
                             gsem[slot])

        def wait_gather(blk, slot):
            pltpu.make_async_copy(t2_hbm.at[idx_all.at[blk]],
                                  rows_v.at[slot], gsem[slot]).wait()

        rot = [(ii + r) & 15 for r in range(_L)]
        rowadd = [r >> 3 for r in rot]
        coladd = [(r & 7) << 7 for r in rot]

        def transpose_block(blk, slot):
            @plsc.parallel_loop(0, 32, 1, unroll=1)
            def per_tile(t):
                bg = t // 4
                eg = t % 4
                bvec = ii + bg * _L
                ebase = jnp.full((_L,), 0, jnp.int32) + eg * _L
                eg2 = eg * 2
                for r in range(_L):
                    evec = ebase + rot[r]
                    v = plsc.load_gather(rows_v.at[slot], [bvec, evec])
                    plsc.store_scatter(tbuf.at[slot],
                                       [rowadd[r] + eg2, coladd[r] + bvec],
                                       v * 8.0)

        def fire_out(blk, slot):
            pltpu.async_copy(tbuf.at[slot], out_hbm.at[blk, :, w], osem[slot])

        def wait_out(blk, slot):
            pltpu.make_async_copy(tbuf.at[slot], out_hbm.at[blk, :, w],
                                  osem[slot]).wait()

        fire_gather(0, 0)
        fire_gather(1, 1)

        def step(blk, slot, pslot):
            @pl.when(blk + 2 < _SEQ)
            def _():
                @pl.when(blk + 2 >= _NSLOT)
                def _():
                    wait_out(blk + 2 - _NSLOT, pslot)

                fire_gather(blk + 2, pslot)

            wait_gather(blk, slot)
            transpose_block(blk, slot)
            fire_out(blk, slot)

        def outer(o, carry):
            for k_ in range(_NSLOT):
                blk = o * _NSLOT + k_
                step(blk, k_, (k_ + 2) % _NSLOT)
            return carry

        lax.fori_loop(0, _SEQ // _NSLOT, outer, 0)
        step(_SEQ - 2, (_SEQ - 2) % _NSLOT, _SEQ % _NSLOT)
        step(_SEQ - 1, (_SEQ - 1) % _NSLOT, (_SEQ + 1) % _NSLOT)

        for k_ in range(_NSLOT):
            wait_out(_SEQ - _NSLOT + k_, (_SEQ - _NSLOT + k_) % _NSLOT)

    return k(x2, t2)


def kernel(x, table):
    tpad = jnp.pad(table, ((0, 0), (0, 2 * _D - table.shape[1])))
    out4 = _embed_lookup(x.reshape(_SEQ * _NBT, _BT), tpad)
    out = (out4.reshape(_SEQ, _D // 8, _NBT, 8, _BT)
           .transpose(2, 4, 0, 1, 3)
           .reshape(_BATCH, _SEQ, _D))
    return out
